# baseline (speedup 1.0000x reference)
import functools

import jax
import jax.numpy as jnp
from jax import lax
from jax.experimental import pallas as pl
from jax.experimental.pallas import tpu as pltpu
from jax.experimental.pallas import tpu_sc as plsc

N = 50000
F_IN = 128
D = 64
T = 1000
H = 2

BN = 1000
NSTEP = N // BN
NBUF = 4
TP = 1024
CHUNKS = ((0, 896), (896, 104))

_SC_B = 1024


def _sc_gather(emb_table, ids_pad):
    mesh = plsc.VectorSubcoreMesh(core_axis_name="c", subcore_axis_name="s")
    info = plsc.get_sparse_core_info()
    nc, ns = info.num_cores, info.num_subcores
    nw = nc * ns
    b_per_w = _SC_B // nw

    @functools.partial(
        pl.kernel, mesh=mesh,
        out_type=jax.ShapeDtypeStruct((_SC_B, 128), jnp.float32),
        scratch_types=[
            pltpu.VMEM((b_per_w,), jnp.int32),
            pltpu.VMEM((b_per_w, 128), jnp.float32),
            pltpu.SemaphoreType.DMA,
        ],
    )
    def k(table_hbm, idx_hbm, out_hbm, idx_v, rows_v, sem):
        wid = lax.axis_index("s") * nc + lax.axis_index("c")
        base = wid * b_per_w
        pltpu.sync_copy(idx_hbm.at[pl.ds(base, b_per_w)], idx_v)
        pltpu.async_copy(table_hbm.at[idx_v], rows_v, sem).wait()
        pltpu.sync_copy(rows_v, out_hbm.at[pl.ds(base, b_per_w)])

    return k(jnp.pad(emb_table, ((0, 0), (0, 128 - D))), ids_pad)


def _leaky_relu(x, slope=0.2):
    return jnp.where(x > 0, x, slope * x)


def _gat_kernel(te_ref, adj_ref, w0_ref, as0_ref, ad0_ref,
                w1_ref, as1_ref, ad1_ref, h_out_ref):
    adj = adj_ref[...]
    te = te_ref[...][:T, :D]

    def layer(h_in, w_ref, asrc_ref, adst_ref):
        acc = jnp.zeros((T, D), dtype=jnp.float32)
        for head in range(H):
            hh = h_in * w_ref[head, :][None, :]
            f_src = jnp.sum(hh * asrc_ref[head, :][None, :], axis=1,
                            keepdims=True)
            f_dst = jnp.sum(hh * adst_ref[head, :][None, :], axis=1,
                            keepdims=True)
            e = f_src + f_dst.T
            e = _leaky_relu(e)
            e = jnp.where(adj > 0, e, jnp.float32(-1e9))
            m = jnp.max(e, axis=1, keepdims=True)
            p = jnp.exp(e - m)
            s = jnp.sum(p, axis=1, keepdims=True)
            a = p / s
            acc = acc + jax.lax.dot_general(
                a, hh, (((1,), (0,)), ((), ())),
                preferred_element_type=jnp.float32)
        return acc * jnp.float32(1.0 / H)

    h = layer(te, w0_ref, as0_ref, ad0_ref)
    h = jnp.where(h > 0, h, jnp.exp(h) - 1.0)
    h = layer(h, w1_ref, as1_ref, ad1_ref)
    h_out_ref[...] = h


def _fused_kernel(x_ref, w1_ref, b1_ref, hp_ref, out_ref, *rest):
    accs = rest[:len(CHUNKS)]
    sems = rest[len(CHUNKS)]
    i = pl.program_id(0)
    s = jax.lax.rem(i, NBUF)
    rows = pl.ds(i * BN, BN)

    @pl.when(i >= NBUF)
    def _():
        for k, (c0, w) in enumerate(CHUNKS):
            pltpu.make_async_copy(accs[k].at[s], out_ref.at[rows, pl.ds(c0, w)],
                                  sems.at[k, s]).wait()

    ent = jax.lax.dot_general(x_ref[...], w1_ref[...],
                              (((1,), (1,)), ((), ())),
                              preferred_element_type=jnp.float32)
    ent = jnp.maximum(ent + b1_ref[...], 0.0)
    logits = jax.lax.dot_general(ent, hp_ref[...],
                                 (((1,), (1,)), ((), ())),
                                 preferred_element_type=jnp.float32)
    for k, (c0, w) in enumerate(CHUNKS):
        accs[k][s] = logits[:, c0:c0 + w]

    for k, (c0, w) in enumerate(CHUNKS):
        pltpu.make_async_copy(accs[k].at[s], out_ref.at[rows, pl.ds(c0, w)],
                              sems.at[k, s]).start()

    @pl.when(i == NSTEP - 1)
    def _():
        for k, (c0, w) in enumerate(CHUNKS):
            for b in range(NBUF):
                pltpu.make_async_copy(accs[k].at[b],
                                      out_ref.at[rows, pl.ds(c0, w)],
                                      sems.at[k, b]).wait()


def _gat(te_pad, type_adj, gw0, ga_src0, ga_dst0, gw1, ga_src1, ga_dst1):
    args = (te_pad, type_adj,
            gw0.reshape(H, D), ga_src0.reshape(H, D), ga_dst0.reshape(H, D),
            gw1.reshape(H, D), ga_src1.reshape(H, D), ga_dst1.reshape(H, D))
    return pl.pallas_call(
        _gat_kernel,
        out_shape=jax.ShapeDtypeStruct((T, D), jnp.float32),
    )(*args)


def _fused(x, W1, b1, h):
    hp = jnp.pad(h, ((0, TP - T), (0, 0)))
    return pl.pallas_call(
        _fused_kernel,
        grid=(NSTEP,),
        in_specs=[
            pl.BlockSpec((BN, F_IN), lambda i: (i, 0)),
            pl.BlockSpec((D, F_IN), lambda i: (0, 0)),
            pl.BlockSpec((1, D), lambda i: (0, 0)),
            pl.BlockSpec((TP, D), lambda i: (0, 0)),
        ],
        out_specs=pl.BlockSpec(memory_space=pltpu.HBM),
        out_shape=jax.ShapeDtypeStruct((N, T), jnp.float32),
        scratch_shapes=(
            [pltpu.VMEM((NBUF, BN, w), jnp.float32) for _, w in CHUNKS]
            + [pltpu.SemaphoreType.DMA((len(CHUNKS), NBUF))]
        ),
        compiler_params=pltpu.CompilerParams(
            dimension_semantics=("arbitrary",),
        ),
    )(x, W1, b1.reshape(1, D), hp)


@jax.jit
def kernel(x, type_ids, type_adj, W1, b1, emb_table, gw0, ga_src0, ga_dst0,
           gw1, ga_src1, ga_dst1):
    ids_pad = jnp.pad(type_ids, (0, _SC_B - T))
    te_pad = _sc_gather(emb_table, ids_pad)
    h = _gat(te_pad, type_adj, gw0, ga_src0, ga_dst0, gw1, ga_src1, ga_dst1)
    return _fused(x, W1, b1, h)
